# baseline (speedup 1.0000x reference)
import functools

import jax
import jax.numpy as jnp
from jax import lax
from jax.experimental import pallas as pl
from jax.experimental.pallas import tpu as pltpu
from jax.experimental.pallas import tpu_sc as plsc

VOCAB = 100000
CHAR_DIM = 64
BATCH = 4096
SEQ_LEN = 200

_LANE = 128
_SPAD = 256
_NW = 32
_B_W = BATCH // _NW
_U = 2
_NBLK = _B_W // _U


@functools.partial(
    pl.kernel,
    out_type=jax.ShapeDtypeStruct((BATCH, SEQ_LEN, 2 * CHAR_DIM), jnp.float32),
    mesh=plsc.VectorSubcoreMesh(core_axis_name="c", subcore_axis_name="s"),
    scratch_types=[
        pltpu.VMEM((2, _U, 2, _LANE), jnp.int32),
        pltpu.VMEM((2, _U, _SPAD, CHAR_DIM), jnp.float32),
        pltpu.SemaphoreType.DMA,
        pltpu.SemaphoreType.DMA,
        pltpu.SemaphoreType.DMA,
    ],
    compiler_params=pltpu.CompilerParams(use_tc_tiling_on_sc=False),
)
def _emb_gather(idx_hbm, tab_hbm, out_hbm, idx_v, rows_v, sem_i, sem_g, sem_o):
    num_cores = 2
    wid = lax.axis_index("s") * num_cores + lax.axis_index("c")
    base = wid * _B_W
    last = base + (_NBLK - 1) * _U

    pltpu.sync_copy(idx_hbm.at[pl.ds(base, _U)], idx_v.at[0])
    pltpu.async_copy(idx_hbm.at[pl.ds(base + _U, _U)], idx_v.at[1], sem_i)

    @pl.loop(0, _NBLK // 2)
    def _pair(p):
        for ph in range(2):
            cur, nxt = ph, 1 - ph
            b0 = base + (2 * p + ph) * _U
            gathers = [
                pltpu.async_copy(
                    tab_hbm.at[idx_v.at[cur].at[u].at[j]],
                    rows_v.at[cur].at[u].at[pl.ds(j * _LANE, _LANE)],
                    sem_g,
                )
                for u in range(_U)
                for j in range(2)
            ]
            pltpu.make_async_copy(
                idx_hbm.at[pl.ds(base, _U)], idx_v.at[nxt], sem_i
            ).wait()
            for c in gathers:
                c.wait()
            b2 = jnp.minimum(b0 + 2 * _U, last)
            pltpu.async_copy(idx_hbm.at[pl.ds(b2, _U)], idx_v.at[cur], sem_i)

            @pl.when(b0 > base)
            def _():
                for u in range(_U):
                    pltpu.make_async_copy(
                        rows_v.at[nxt].at[u].at[pl.ds(0, SEQ_LEN)],
                        out_hbm.at[base].at[:, pl.ds(0, CHAR_DIM)],
                        sem_o,
                    ).wait()

            for u in range(_U):
                pltpu.async_copy(
                    rows_v.at[cur].at[u].at[pl.ds(0, SEQ_LEN)],
                    out_hbm.at[b0 + u].at[:, pl.ds(0, CHAR_DIM)],
                    sem_o,
                )

    for u in range(_U):
        pltpu.make_async_copy(
            rows_v.at[1].at[u].at[pl.ds(0, SEQ_LEN)],
            out_hbm.at[base].at[:, pl.ds(0, CHAR_DIM)],
            sem_o,
        ).wait()
    pltpu.make_async_copy(idx_hbm.at[pl.ds(base, _U)], idx_v.at[0], sem_i).wait()


def kernel(x, char_lookup):
    xpad = jnp.pad(x.astype(jnp.int32), ((0, 0), (0, _SPAD - SEQ_LEN)))
    idx = xpad.reshape(BATCH, 2, _LANE)
    out_padded = _emb_gather(idx, char_lookup)
    return out_padded[:, :, :CHAR_DIM]
